# baseline (speedup 1.0000x reference)
import functools

import jax
import jax.numpy as jnp
from jax import lax
from jax.experimental import pallas as pl
from jax.experimental.pallas import tpu as pltpu
from jax.experimental.pallas import tpu_sc as plsc

S = 200
B = 4096
D = 64
BLK = 128
NW = 32
N_BLKS = B // BLK
ITEMS = S * N_BLKS
PER_W = ITEMS // NW


def _emb_kernel(rowp_hbm, par_hbm, wp_hbm, out_hbm, rowp, par64, rows, tr,
                *sems):
    isem = sems[0:2]
    gsem = sems[2:4]
    wsem = sems[4:6]
    w = lax.axis_index("s") * 2 + lax.axis_index("c")
    base = w * PER_W
    iota16 = lax.iota(jnp.int32, 16)

    def item_sb(k):
        kg = base + k
        return kg // N_BLKS, (kg % N_BLKS) * BLK

    def stage(k, b):
        s, b0 = item_sb(k)
        pltpu.async_copy(rowp_hbm.at[s, pl.ds(b0, BLK)], rowp.at[b], isem[b])
        pltpu.async_copy(par_hbm.at[s, pl.ds(b0, BLK)], par64.at[b], isem[b])

    def wait_stage(b):
        pltpu.make_async_copy(
            rowp_hbm.at[0, pl.ds(0, BLK)], rowp.at[b], isem[b]
        ).wait()
        pltpu.make_async_copy(
            par_hbm.at[0, pl.ds(0, BLK)], par64.at[b], isem[b]
        ).wait()

    def fire_gather(b):
        pltpu.async_copy(wp_hbm.at[rowp.at[b]], rows.at[b], gsem[b])

    def wait_gather(b):
        pltpu.make_async_copy(
            wp_hbm.at[rowp.at[b]], rows.at[b], gsem[b]
        ).wait()

    def transpose_item(b):
        rowsb = rows.at[b]
        trb = tr.at[b]
        rowg = [g * 16 + iota16 for g in range(8)]
        parg = [par64.at[b][pl.ds(g * 16, 16)] for g in range(8)]

        def dbody(d, c):
            dv = jnp.zeros((16,), jnp.int32) + d
            for g in range(8):
                v = plsc.load_gather(rowsb, [rowg[g], parg[g] + d])
                plsc.store_scatter(trb, [dv, rowg[g]], v)
            return c

        lax.fori_loop(0, D, dbody, 0, unroll=2)

    def fire_wb(k, b):
        s, b0 = item_sb(k)
        pltpu.async_copy(tr.at[b], out_hbm.at[s, :, pl.ds(b0, BLK)], wsem[b])

    def wait_wb(b):
        pltpu.make_async_copy(
            tr.at[b], out_hbm.at[0, :, pl.ds(0, BLK)], wsem[b]
        ).wait()

    def item_step(k, b, first=False, fire_next=True, stage_ahead=True):
        nb = 1 - b
        if fire_next:
            wait_stage(nb)
            fire_gather(nb)
        wait_gather(b)
        if not first:
            wait_wb(b)
        transpose_item(b)
        fire_wb(k, b)
        if stage_ahead:
            stage(k + 2, b)

    stage(0, 0)
    wait_stage(0)
    fire_gather(0)
    stage(1, 1)

    item_step(0, 0, first=True)
    item_step(1, 1, first=True)

    def round_body(r, carry):
        for half in (0, 1):
            item_step(2 * r + half, half)
        return carry

    lax.fori_loop(1, PER_W // 2 - 1, round_body, 0)

    item_step(PER_W - 2, 0, stage_ahead=False)
    item_step(PER_W - 1, 1, fire_next=False, stage_ahead=False)
    wait_wb(0)
    wait_wb(1)


def kernel(token_ids, weight):
    tid = token_ids.astype(jnp.int32)
    rowp_t = (tid >> 1).T
    par_t = ((tid & 1) << 6).T
    wp = weight.reshape(500000, 128)
    mesh = plsc.VectorSubcoreMesh(core_axis_name="c", subcore_axis_name="s")
    run = functools.partial(
        pl.kernel,
        mesh=mesh,
        compiler_params=pltpu.CompilerParams(needs_layout_passes=False),
        out_type=jax.ShapeDtypeStruct((S, D, B), jnp.float32),
        scratch_types=[
            pltpu.VMEM((2, BLK), jnp.int32),
            pltpu.VMEM((2, BLK), jnp.int32),
            pltpu.VMEM((2, BLK, 128), jnp.float32),
            pltpu.VMEM((2, D, BLK), jnp.float32),
        ]
        + [pltpu.SemaphoreType.DMA] * 6,
    )(_emb_kernel)
    out3 = run(rowp_t, par_t, wp)
    return out3.transpose(2, 0, 1)
